# Initial kernel scaffold; baseline (speedup 1.0000x reference)
import jax
import jax.numpy as jnp
from jax.experimental import pallas as pl


def kernel(pred_dsmat, gt_perm, src_ns, tgt_ns):
    raise NotImplementedError("write your pallas kernel here")


# speedup vs baseline: 47.8264x; 47.8264x over previous
import functools

import jax
import jax.numpy as jnp
from jax.experimental import pallas as pl
from jax.experimental.pallas import tpu as pltpu

_B, _N1, _N2 = 16, 1024, 1024
_R = 256
_NB = _N1 // _R


def _cw_body(src_ref, tgt_ref, pred_ref, gt_ref, out_ref, acc_ref):
    b = pl.program_id(0)
    r = pl.program_id(1)
    ns = src_ref[b]
    nt = tgt_ref[b]
    px = pred_ref[0]
    gx = gt_ref[0]
    col = jax.lax.broadcasted_iota(jnp.int32, (_R, _N2), 1)
    valid = col < nt
    neg = jnp.float32(-jnp.inf)

    mg = jnp.where(valid, gx, neg)
    g1 = jnp.max(mg, axis=1, keepdims=True)
    gt_idx = jnp.min(jnp.where(mg == g1, col, _N2 * 2), axis=1, keepdims=True)
    pred_at_gt = jnp.max(jnp.where(col == gt_idx, px, neg), axis=1, keepdims=True)

    mp = jnp.where(valid, px, neg)
    m1 = jnp.max(mp, axis=1, keepdims=True)
    i1 = jnp.max(jnp.where(mp == m1, col, -1), axis=1, keepdims=True)
    m2 = jnp.max(jnp.where(col == i1, neg, mp), axis=1, keepdims=True)

    sel = jnp.where(i1 == gt_idx, m2, m1)
    contrib = jnp.log(sel) - jnp.log(pred_at_gt)
    row = r * _R + jax.lax.broadcasted_iota(jnp.int32, (_R, 1), 0)
    partial = jnp.sum(jnp.where(row < ns, contrib, 0.0))

    is_first = jnp.logical_and(b == 0, r == 0)
    acc_ref[0] = jnp.where(is_first, 0.0, acc_ref[0]) + partial

    @pl.when(jnp.logical_and(b == _B - 1, r == _NB - 1))
    def _():
        n_sum = jax.lax.fori_loop(
            0, _B, lambda i, s: s + src_ref[i].astype(jnp.float32), jnp.float32(0.0)
        )
        out_ref[0, 0] = acc_ref[0] / n_sum


@functools.partial(jax.jit, static_argnames=("interpret",))
def kernel(pred_dsmat, gt_perm, src_ns, tgt_ns, interpret=False):
    pred_dsmat = pred_dsmat.astype(jnp.float32)
    gt_perm = gt_perm.astype(jnp.float32)
    out = pl.pallas_call(
        _cw_body,
        grid=(_B, _NB),
        in_specs=[
            pl.BlockSpec(memory_space=pltpu.SMEM),
            pl.BlockSpec(memory_space=pltpu.SMEM),
            pl.BlockSpec((1, _R, _N2), lambda b, r: (b, r, 0)),
            pl.BlockSpec((1, _R, _N2), lambda b, r: (b, r, 0)),
        ],
        out_specs=pl.BlockSpec(memory_space=pltpu.SMEM),
        out_shape=jax.ShapeDtypeStruct((1, 1), jnp.float32),
        scratch_shapes=[pltpu.SMEM((1,), jnp.float32)],
        interpret=interpret,
    )(src_ns, tgt_ns, pred_dsmat, gt_perm)
    return out[0, 0]


# speedup vs baseline: 48.9446x; 1.0234x over previous
import functools

import jax
import jax.numpy as jnp
from jax.experimental import pallas as pl
from jax.experimental.pallas import tpu as pltpu

_B, _N1, _N2 = 16, 1024, 1024
_R = 256
_NB = _N1 // _R


def _cw_body(src_ref, tgt_ref, pred_ref, gt_ref, out_ref, acc_ref):
    b = pl.program_id(0)
    r = pl.program_id(1)
    ns = src_ref[b]
    nt = tgt_ref[b]
    px = pred_ref[0]
    gx = gt_ref[0]
    col = jax.lax.broadcasted_iota(jnp.int32, (_R, _N2), 1)
    valid = col < nt
    neg = jnp.float32(-jnp.inf)

    mg = jnp.where(valid, gx, neg)
    g1 = jnp.max(mg, axis=1, keepdims=True)
    gt_idx = jnp.min(jnp.where(mg == g1, col, _N2 * 2), axis=1, keepdims=True)
    pred_at_gt = jnp.max(jnp.where(col == gt_idx, px, neg), axis=1, keepdims=True)

    mp = jnp.where(valid, px, neg)
    h = jnp.maximum(mp[:, :512], mp[:, 512:])
    l = jnp.minimum(mp[:, :512], mp[:, 512:])
    for w in (256, 128):
        h1, h2 = h[:, :w], h[:, w:]
        l_new = jnp.maximum(jnp.minimum(h1, h2), jnp.maximum(l[:, :w], l[:, w:]))
        h = jnp.maximum(h1, h2)
        l = l_new
    m1 = jnp.max(h, axis=1, keepdims=True)
    is_m1 = h == m1
    m2h_strict = jnp.max(jnp.where(is_m1, neg, h), axis=1, keepdims=True)
    dup = jnp.sum(jnp.where(is_m1, 1, 0), axis=1, keepdims=True) > 1
    m2h = jnp.where(dup, m1, m2h_strict)
    m2 = jnp.maximum(m2h, jnp.max(l, axis=1, keepdims=True))

    sel = jnp.where(pred_at_gt == m1, m2, m1)
    contrib = jnp.log(sel) - jnp.log(pred_at_gt)
    row = r * _R + jax.lax.broadcasted_iota(jnp.int32, (_R, 1), 0)
    partial = jnp.sum(jnp.where(row < ns, contrib, 0.0))

    is_first = jnp.logical_and(b == 0, r == 0)
    acc_ref[0] = jnp.where(is_first, 0.0, acc_ref[0]) + partial

    @pl.when(jnp.logical_and(b == _B - 1, r == _NB - 1))
    def _():
        n_sum = jax.lax.fori_loop(
            0, _B, lambda i, s: s + src_ref[i].astype(jnp.float32), jnp.float32(0.0)
        )
        out_ref[0, 0] = acc_ref[0] / n_sum


@functools.partial(jax.jit, static_argnames=("interpret",))
def kernel(pred_dsmat, gt_perm, src_ns, tgt_ns, interpret=False):
    pred_dsmat = pred_dsmat.astype(jnp.float32)
    gt_perm = gt_perm.astype(jnp.float32)
    out = pl.pallas_call(
        _cw_body,
        grid=(_B, _NB),
        in_specs=[
            pl.BlockSpec(memory_space=pltpu.SMEM),
            pl.BlockSpec(memory_space=pltpu.SMEM),
            pl.BlockSpec((1, _R, _N2), lambda b, r: (b, r, 0)),
            pl.BlockSpec((1, _R, _N2), lambda b, r: (b, r, 0)),
        ],
        out_specs=pl.BlockSpec(memory_space=pltpu.SMEM),
        out_shape=jax.ShapeDtypeStruct((1, 1), jnp.float32),
        scratch_shapes=[pltpu.SMEM((1,), jnp.float32)],
        interpret=interpret,
    )(src_ns, tgt_ns, pred_dsmat, gt_perm)
    return out[0, 0]
